# Initial kernel scaffold; baseline (speedup 1.0000x reference)
import jax
import jax.numpy as jnp
from jax.experimental import pallas as pl


def kernel(x_batch, param_slab):
    raise NotImplementedError("write your pallas kernel here")


# speedup vs baseline: 3.4258x; 3.4258x over previous
import jax
import jax.numpy as jnp
from jax.experimental import pallas as pl
from jax.experimental.pallas import tpu as pltpu

_IN = 6
_HID = 32
_OUT = 2
_PACK = 4
_ROW_IN = _PACK * _IN
_ROW_OUT = _PACK * _OUT
_TILE_R = 4096


def _sigmoid(h):
    return pl.reciprocal(1.0 + jnp.exp(-h), approx=True)


def _mlp_kernel(x_ref, p1_ref, w2_ref, o_ref):
    h = jnp.dot(x_ref[...], p1_ref[0:_ROW_IN, :],
                preferred_element_type=jnp.float32)
    a = _sigmoid(h + p1_ref[_ROW_IN:_ROW_IN + 1, :])
    o = jnp.dot(a, w2_ref[...], preferred_element_type=jnp.float32)
    o_ref[...] = _sigmoid(o + p1_ref[_ROW_IN + 1:_ROW_IN + 2, 0:_ROW_OUT])


def kernel(x_batch, param_slab):
    B = x_batch.shape[0]
    span = _PACK * _TILE_R
    b_pad = pl.cdiv(B, span) * span
    if b_pad != B:
        x_batch = jnp.pad(x_batch.astype(jnp.float32),
                          ((0, b_pad - B), (0, 0)))
    rows = b_pad // _PACK
    x4 = x_batch.reshape(rows, _ROW_IN)

    w1 = param_slab[0:_IN, 0:_HID]
    b1 = param_slab[8, 0:_HID]
    w2 = param_slab[9:9 + _OUT, 0:_HID].T
    b2 = param_slab[11, 0:_OUT]
    eye = jnp.eye(_PACK, dtype=jnp.float32)
    p1 = jnp.zeros((32, 128), jnp.float32)
    p1 = p1.at[0:_ROW_IN, :].set(jnp.kron(eye, w1))
    p1 = p1.at[_ROW_IN, :].set(jnp.tile(b1, _PACK))
    p1 = p1.at[_ROW_IN + 1, 0:_ROW_OUT].set(jnp.tile(b2, _PACK))
    w2d = jnp.kron(eye, w2)

    n_tiles = rows // _TILE_R
    out = pl.pallas_call(
        _mlp_kernel,
        out_shape=jax.ShapeDtypeStruct((rows, _ROW_OUT), jnp.float32),
        grid=(n_tiles,),
        in_specs=[
            pl.BlockSpec((_TILE_R, _ROW_IN), lambda i: (i, 0)),
            pl.BlockSpec((32, 128), lambda i: (0, 0)),
            pl.BlockSpec((128, _ROW_OUT), lambda i: (0, 0)),
        ],
        out_specs=pl.BlockSpec((_TILE_R, _ROW_OUT), lambda i: (i, 0)),
        compiler_params=pltpu.CompilerParams(
            dimension_semantics=("parallel",)),
    )(x4, p1, w2d)
    return out.reshape(b_pad, _OUT)[:B]


# speedup vs baseline: 4.1256x; 1.2043x over previous
import jax
import jax.numpy as jnp
from jax.experimental import pallas as pl
from jax.experimental.pallas import tpu as pltpu

_IN = 6
_HID = 32
_OUT = 2
_TILE_B = 8192


def _mlp_kernel(x_ref, p_ref, w2_ref, o_ref):
    h = jnp.dot(x_ref[...], p_ref[0:_IN, :],
                preferred_element_type=jnp.float32)
    t = jnp.tanh(h + p_ref[8:9, :])
    u = jnp.tanh(jnp.dot(t, w2_ref[...],
                         preferred_element_type=jnp.float32)
                 + p_ref[9:10, 0:_OUT])
    o_ref[...] = 0.5 * u + 0.5


def kernel(x_batch, param_slab):
    B = x_batch.shape[0]
    b_pad = pl.cdiv(B, _TILE_B) * _TILE_B
    if b_pad != B:
        x_batch = jnp.pad(x_batch.astype(jnp.float32),
                          ((0, b_pad - B), (0, 0)))

    w2 = param_slab[9:9 + _OUT, 0:_HID].T
    b2 = param_slab[11, 0:_OUT]
    p = param_slab.at[0:8, :].multiply(0.5)
    p = p.at[8, :].multiply(0.5)
    p = p.at[9, :].set(0.0)
    p = p.at[9, 0:_OUT].set(0.5 * (b2 + 0.5 * jnp.sum(w2, axis=0)))
    w2f = jnp.zeros((128, _OUT), jnp.float32)
    w2f = w2f.at[0:_HID, :].set(0.25 * w2)

    n_tiles = b_pad // _TILE_B
    out = pl.pallas_call(
        _mlp_kernel,
        out_shape=jax.ShapeDtypeStruct((b_pad, _OUT), jnp.float32),
        grid=(n_tiles,),
        in_specs=[
            pl.BlockSpec((_TILE_B, _IN), lambda i: (i, 0)),
            pl.BlockSpec((16, 128), lambda i: (0, 0)),
            pl.BlockSpec((128, _OUT), lambda i: (0, 0)),
        ],
        out_specs=pl.BlockSpec((_TILE_B, _OUT), lambda i: (i, 0)),
        compiler_params=pltpu.CompilerParams(
            dimension_semantics=("parallel",)),
    )(x_batch, p, w2f)
    return out[:B]
